# Initial kernel scaffold; baseline (speedup 1.0000x reference)
import jax
import jax.numpy as jnp
from jax.experimental import pallas as pl


def kernel(pos1, pos2, feature1, feature2, nsample, pos1_raw, pe_w1, pe_b1, pe_w2, pe_b2, qk_w1, qk_b1, qk_w2, qk_b2, cv_w1, cv_b1, cv_w2, cv_b2, mlp_w, mlp_b):
    raise NotImplementedError("write your pallas kernel here")


# speedup vs baseline: 13.7689x; 13.7689x over previous
import functools
import math

import jax
import jax.numpy as jnp
from jax import lax
from jax.experimental import pallas as pl
from jax.experimental.pallas import tpu as pltpu
from jax.experimental.pallas import tpu_sc as plsc

BN = 512
RAD2 = 2.5 * 2.5

_DN = (((1,), (1,)), ((), ()))


def _dot(x, w):
    return lax.dot_general(x, w, _DN, preferred_element_type=jnp.float32)


def _feats_body(pos_ref, feat_ref, pw1, pb1, pw2, pb2, qw1, qb1, qw2, qb2,
                out_ref):
    pos = pos_ref[...]
    feat = feat_ref[...]
    h = jnp.maximum(_dot(pos, pw1[...]) + pb1[...], 0.0)
    emb = _dot(h, pw2[...]) + pb2[...]
    f = jnp.concatenate([feat, emb], axis=1)
    h2 = jnp.maximum(_dot(f, qw1[...]) + qb1[...], 0.0)
    out_ref[...] = _dot(h2, qw2[...]) + qb2[...]


def _knn_body(q_ref, rcm_ref, out_ref, *, k):
    q = q_ref[...]
    rcm = rcm_ref[...]
    qq = jnp.sum(q * q, axis=1, keepdims=True)
    rr = jnp.sum(rcm * rcm, axis=0, keepdims=True)
    qr = lax.dot_general(q, rcm, (((1,), (0,)), ((), ())),
                         preferred_element_type=jnp.float32)
    d2 = qq + rr - 2.0 * qr
    iota = lax.broadcasted_iota(jnp.int32, d2.shape, 1)
    cols = []
    for _ in range(k):
        am = jnp.argmin(d2, axis=1).astype(jnp.int32)
        cols.append(am)
        d2 = jnp.where(iota == am[:, None], jnp.inf, d2)
    out_ref[...] = jnp.stack(cols, axis=1)


def _curv_one(q, rcm):
    qq = jnp.sum(q * q, axis=1, keepdims=True)
    rr = jnp.sum(rcm * rcm, axis=0, keepdims=True)
    qr = lax.dot_general(q, rcm, (((1,), (0,)), ((), ())),
                         preferred_element_type=jnp.float32)
    d2 = qq + rr - 2.0 * qr
    iota = lax.broadcasted_iota(jnp.int32, d2.shape, 1)
    acc = jnp.zeros((q.shape[0], 3), jnp.float32)
    cnt = jnp.zeros((q.shape[0], 1), jnp.float32)
    for _ in range(10):
        mv = jnp.min(d2, axis=1, keepdims=True)
        am = jnp.argmin(d2, axis=1).astype(jnp.int32)
        within = mv <= RAD2
        onehot = ((iota == am[:, None]) & within).astype(jnp.float32)
        acc = acc + lax.dot_general(onehot, rcm, (((1,), (1,)), ((), ())),
                                    preferred_element_type=jnp.float32)
        cnt = cnt + within.astype(jnp.float32)
        d2 = jnp.where(iota == am[:, None], jnp.inf, d2)
    return (acc - cnt * q) / 9.0


def _curv_body(q1_ref, r1_ref, q2_ref, r2_ref, cw1, cb1, cw2, cb2, out_ref):
    def cnet(x):
        h = jnp.maximum(_dot(x, cw1[...]) + cb1[...], 0.0)
        return _dot(h, cw2[...]) + cb2[...]

    c1 = cnet(_curv_one(q1_ref[...], r1_ref[...]))
    c2 = cnet(_curv_one(q2_ref[...], r2_ref[...]))
    out_ref[...] = (c1 - c2) ** 2


def _final_body(cost_ref, curv_ref, w1, w2, b, out_ref):
    out_ref[...] = (_dot(w1[...], cost_ref[...])
                    + _dot(w2[...], curv_ref[...]) + b[...])


def _make_sc_attn(total_pts, d, scale):
    NC, NS = 2, 16
    NW = NC * NS
    ppw = total_pts // NW
    CH = 32
    nchunk = ppw // CH
    rows_per_chunk = CH * 16
    ncg = d // 16

    mesh = plsc.VectorSubcoreMesh(core_axis_name="c", subcore_axis_name="s")

    @functools.partial(
        pl.kernel, mesh=mesh,
        out_type=jax.ShapeDtypeStruct((total_pts, d), jnp.float32),
        scratch_types=[
            pltpu.VMEM((rows_per_chunk,), jnp.int32),
            pltpu.VMEM((rows_per_chunk, d), jnp.float32),
            pltpu.VMEM((CH, d), jnp.float32),
            pltpu.VMEM((CH, d), jnp.float32),
            pltpu.SemaphoreType.DMA,
        ],
    )
    def sc_attn(f2_hbm, f1_hbm, idx_hbm, out_hbm, idx_v, rows_v, f1_v, out_v,
                sem):
        wid = lax.axis_index("s") * NC + lax.axis_index("c")

        def chunk_body(ci, carry):
            pbase = wid * ppw + ci * CH
            pltpu.sync_copy(idx_hbm.at[pl.ds(pbase * 16, rows_per_chunk)],
                            idx_v)
            copies = [
                pltpu.async_copy(f2_hbm.at[idx_v.at[pl.ds(j * 128, 128)]],
                                 rows_v.at[pl.ds(j * 128, 128)], sem)
                for j in range(rows_per_chunk // 128)
            ]
            for c in copies:
                c.wait()
            pltpu.sync_copy(f1_hbm.at[pl.ds(pbase, CH)], f1_v)

            def point_body(p, carry2):
                for cg in range(ncg):
                    c0 = cg * 16
                    f1v = f1_v[p, pl.ds(c0, 16)]
                    f1s = f1v * scale
                    f2s = [rows_v[p * 16 + k, pl.ds(c0, 16)]
                           for k in range(16)]
                    a = [f1s * f2 for f2 in f2s]
                    m = a[0]
                    for t in a[1:]:
                        m = jnp.maximum(m, t)
                    e = [jnp.exp(t - m) for t in a]
                    s = e[0]
                    for t in e[1:]:
                        s = s + t
                    num = jnp.zeros((16,), jnp.float32)
                    for k in range(16):
                        dd = f1v - f2s[k]
                        num = num + e[k] * dd * dd
                    out_v[p, pl.ds(c0, 16)] = num / s
                return carry2

            lax.fori_loop(0, CH, point_body, 0)
            pltpu.sync_copy(out_v, out_hbm.at[pl.ds(pbase, CH)])
            return carry

        lax.fori_loop(0, nchunk, chunk_body, 0)

    return sc_attn


def kernel(pos1, pos2, feature1, feature2, nsample, pos1_raw, pe_w1, pe_b1,
           pe_w2, pe_b2, qk_w1, qk_b1, qk_w2, qk_b2, cv_w1, cv_b1, cv_w2,
           cv_b2, mlp_w, mlp_b):
    B, _, N = pos1.shape
    C = feature1.shape[1]
    D = qk_w1.shape[0]
    NB = N // BN

    pos1_r = jnp.transpose(pos1, (0, 2, 1))
    pos2_r = jnp.transpose(pos2, (0, 2, 1))
    pos1_raw_r = jnp.transpose(pos1_raw, (0, 2, 1))
    feat1_r = jnp.transpose(feature1, (0, 2, 1))
    feat2_r = jnp.transpose(feature2, (0, 2, 1))

    pe_b1_2 = pe_b1.reshape(1, -1)
    pe_b2_2 = pe_b2.reshape(1, -1)
    qk_b1_2 = qk_b1.reshape(1, -1)
    qk_b2_2 = qk_b2.reshape(1, -1)
    cv_b1_2 = cv_b1.reshape(1, -1)
    cv_b2_2 = cv_b2.reshape(1, -1)

    pos_st = jnp.concatenate([pos1_r, pos2_r], axis=0)
    feat_st = jnp.concatenate([feat1_r, feat2_r], axis=0)
    wspec = lambda shp: pl.BlockSpec(shp, lambda i, j: (0, 0))
    frows = pl.pallas_call(
        _feats_body,
        grid=(2 * B, NB),
        in_specs=[
            pl.BlockSpec((None, BN, 3), lambda i, j: (i, j, 0)),
            pl.BlockSpec((None, BN, C), lambda i, j: (i, j, 0)),
            wspec(pe_w1.shape), wspec(pe_b1_2.shape),
            wspec(pe_w2.shape), wspec(pe_b2_2.shape),
            wspec(qk_w1.shape), wspec(qk_b1_2.shape),
            wspec(qk_w2.shape), wspec(qk_b2_2.shape),
        ],
        out_specs=pl.BlockSpec((None, BN, D), lambda i, j: (i, j, 0)),
        out_shape=jax.ShapeDtypeStruct((2 * B, N, D), jnp.float32),
    )(pos_st, feat_st, pe_w1, pe_b1_2, pe_w2, pe_b2_2, qk_w1, qk_b1_2,
      qk_w2, qk_b2_2)
    f1_rows = frows[:B].reshape(B * N, D)
    f2_rows = frows[B:].reshape(B * N, D)

    idx_loc = pl.pallas_call(
        functools.partial(_knn_body, k=16),
        grid=(B, NB),
        in_specs=[
            pl.BlockSpec((None, BN, 3), lambda b, j: (b, j, 0)),
            pl.BlockSpec((None, 3, N), lambda b, j: (b, 0, 0)),
        ],
        out_specs=pl.BlockSpec((None, BN, 16), lambda b, j: (b, j, 0)),
        out_shape=jax.ShapeDtypeStruct((B, N, 16), jnp.int32),
    )(pos1_r, pos2)

    mult = jnp.asarray(nsample, jnp.int32) // 16
    idx_glob = (jnp.clip(idx_loc * mult, 0, N - 1)
                + (jnp.arange(B, dtype=jnp.int32) * N)[:, None, None])
    idx_flat = idx_glob.reshape(B * N * 16)

    curv_rows = pl.pallas_call(
        _curv_body,
        grid=(B, NB),
        in_specs=[
            pl.BlockSpec((None, BN, 3), lambda b, j: (b, j, 0)),
            pl.BlockSpec((None, 3, N), lambda b, j: (b, 0, 0)),
            pl.BlockSpec((None, BN, 3), lambda b, j: (b, j, 0)),
            pl.BlockSpec((None, 3, N), lambda b, j: (b, 0, 0)),
            wspec(cv_w1.shape), wspec(cv_b1_2.shape),
            wspec(cv_w2.shape), wspec(cv_b2_2.shape),
        ],
        out_specs=pl.BlockSpec((None, BN, 32), lambda b, j: (b, j, 0)),
        out_shape=jax.ShapeDtypeStruct((B, N, 32), jnp.float32),
    )(pos1_raw_r, pos1_raw, pos1_r, pos1, cv_w1, cv_b1_2, cv_w2, cv_b2_2)

    sc_attn = _make_sc_attn(B * N, D, 1.0 / math.sqrt(D))
    cost_rows = sc_attn(f2_rows, f1_rows, idx_flat)

    out = pl.pallas_call(
        _final_body,
        grid=(B, NB),
        in_specs=[
            pl.BlockSpec((None, BN, D), lambda b, j: (b, j, 0)),
            pl.BlockSpec((None, BN, 32), lambda b, j: (b, j, 0)),
            wspec((mlp_w.shape[0], D)),
            wspec((mlp_w.shape[0], 32)),
            wspec((mlp_w.shape[0], 1)),
        ],
        out_specs=pl.BlockSpec((None, mlp_w.shape[0], BN),
                               lambda b, j: (b, 0, j)),
        out_shape=jax.ShapeDtypeStruct((B, mlp_w.shape[0], N), jnp.float32),
    )(cost_rows.reshape(B, N, D), curv_rows, mlp_w[:, :D], mlp_w[:, D:],
      mlp_b.reshape(-1, 1))

    return (pos1, out)


# speedup vs baseline: 16.9100x; 1.2281x over previous
import functools
import math

import jax
import jax.numpy as jnp
from jax import lax
from jax.experimental import pallas as pl
from jax.experimental.pallas import tpu as pltpu
from jax.experimental.pallas import tpu_sc as plsc

BN = 512
CBN = 256
RAD2 = 2.5 * 2.5

_DN = (((1,), (1,)), ((), ()))


def _dot(x, w):
    return lax.dot_general(x, w, _DN, preferred_element_type=jnp.float32)


def _feats_body(pos_ref, feat_ref, pw1, pb1, pw2, pb2, qw1, qb1, qw2, qb2,
                out_ref):
    pos = pos_ref[...]
    feat = feat_ref[...]
    h = jnp.maximum(_dot(pos, pw1[...]) + pb1[...], 0.0)
    emb = _dot(h, pw2[...]) + pb2[...]
    f = jnp.concatenate([feat, emb], axis=1)
    h2 = jnp.maximum(_dot(f, qw1[...]) + qb1[...], 0.0)
    out_ref[...] = _dot(h2, qw2[...]) + qb2[...]


def _knn_body(q_ref, rcm_ref, out_ref, *, k):
    q = q_ref[...]
    rcm = rcm_ref[...]
    qq = jnp.sum(q * q, axis=1, keepdims=True)
    rr = jnp.sum(rcm * rcm, axis=0, keepdims=True)
    qr = lax.dot_general(q, rcm, (((1,), (0,)), ((), ())),
                         preferred_element_type=jnp.float32)
    d2 = qq + rr - 2.0 * qr
    iota = lax.broadcasted_iota(jnp.int32, d2.shape, 1)
    cols = []
    for _ in range(k):
        am = jnp.argmin(d2, axis=1).astype(jnp.int32)
        cols.append(am)
        d2 = jnp.where(iota == am[:, None], jnp.inf, d2)
    out_ref[...] = jnp.stack(cols, axis=1)


def _curv_one(q, rcm):
    qq = jnp.sum(q * q, axis=1, keepdims=True)
    rr = jnp.sum(rcm * rcm, axis=0, keepdims=True)
    qr = lax.dot_general(q, rcm, (((1,), (0,)), ((), ())),
                         preferred_element_type=jnp.float32)
    d2 = qq + rr - 2.0 * qr
    d2 = jnp.where(d2 > RAD2, jnp.inf, d2)
    iota = lax.broadcasted_iota(jnp.int32, d2.shape, 1)
    sel = jnp.zeros(d2.shape, jnp.float32)
    for _ in range(10):
        am = jnp.argmin(d2, axis=1).astype(jnp.int32)
        hit = iota == am[:, None]
        sel = jnp.where(hit & (d2 < jnp.inf), 1.0, sel)
        d2 = jnp.where(hit, jnp.inf, d2)
    r4 = jnp.concatenate([rcm, jnp.ones((1, rcm.shape[1]), jnp.float32)], 0)
    acc4 = lax.dot_general(sel, r4, (((1,), (1,)), ((), ())),
                           preferred_element_type=jnp.float32)
    return (acc4[:, :3] - acc4[:, 3:4] * q) / 9.0


def _curv_body(q_ref, rcm_ref, cw1, cb1, cw2, cb2, out_ref):
    def cnet(x):
        h = jnp.maximum(_dot(x, cw1[...]) + cb1[...], 0.0)
        return _dot(h, cw2[...]) + cb2[...]

    out_ref[...] = cnet(_curv_one(q_ref[...], rcm_ref[...]))


def _final_body(cost_ref, cn1_ref, cn2_ref, w1, w2, b, out_ref):
    curv = (cn1_ref[...] - cn2_ref[...]) ** 2
    out_ref[...] = (_dot(w1[...], cost_ref[...])
                    + _dot(w2[...], curv) + b[...])


def _make_sc_attn(total_pts, d, scale):
    NC, NS = 2, 16
    NW = NC * NS
    ppw = total_pts // NW
    CH = 32
    nchunk = ppw // CH
    rows_per_chunk = CH * 16
    ncg = d // 16

    mesh = plsc.VectorSubcoreMesh(core_axis_name="c", subcore_axis_name="s")

    @functools.partial(
        pl.kernel, mesh=mesh,
        out_type=jax.ShapeDtypeStruct((total_pts, d), jnp.float32),
        scratch_types=[
            pltpu.VMEM((rows_per_chunk,), jnp.int32),
            pltpu.VMEM((rows_per_chunk, d), jnp.float32),
            pltpu.VMEM((CH, d), jnp.float32),
            pltpu.VMEM((CH, d), jnp.float32),
            pltpu.SemaphoreType.DMA,
        ],
    )
    def sc_attn(f2_hbm, f1_hbm, idx_hbm, out_hbm, idx_v, rows_v, f1_v, out_v,
                sem):
        wid = lax.axis_index("s") * NC + lax.axis_index("c")

        def chunk_body(ci, carry):
            pbase = wid * ppw + ci * CH
            pltpu.sync_copy(idx_hbm.at[pl.ds(pbase * 16, rows_per_chunk)],
                            idx_v)
            copies = [
                pltpu.async_copy(f2_hbm.at[idx_v.at[pl.ds(j * 128, 128)]],
                                 rows_v.at[pl.ds(j * 128, 128)], sem)
                for j in range(rows_per_chunk // 128)
            ]
            for c in copies:
                c.wait()
            pltpu.sync_copy(f1_hbm.at[pl.ds(pbase, CH)], f1_v)

            def point_body(p, carry2):
                for cg in range(ncg):
                    c0 = cg * 16
                    f1v = f1_v[p, pl.ds(c0, 16)]
                    f1s = f1v * scale
                    f2s = [rows_v[p * 16 + k, pl.ds(c0, 16)]
                           for k in range(16)]
                    a = [f1s * f2 for f2 in f2s]
                    m = a[0]
                    for t in a[1:]:
                        m = jnp.maximum(m, t)
                    e = [jnp.exp(t - m) for t in a]
                    s = e[0]
                    for t in e[1:]:
                        s = s + t
                    num = jnp.zeros((16,), jnp.float32)
                    for k in range(16):
                        dd = f1v - f2s[k]
                        num = num + e[k] * dd * dd
                    out_v[p, pl.ds(c0, 16)] = num / s
                return carry2

            lax.fori_loop(0, CH, point_body, 0)
            pltpu.sync_copy(out_v, out_hbm.at[pl.ds(pbase, CH)])
            return carry

        lax.fori_loop(0, nchunk, chunk_body, 0)

    return sc_attn


def kernel(pos1, pos2, feature1, feature2, nsample, pos1_raw, pe_w1, pe_b1,
           pe_w2, pe_b2, qk_w1, qk_b1, qk_w2, qk_b2, cv_w1, cv_b1, cv_w2,
           cv_b2, mlp_w, mlp_b):
    B, _, N = pos1.shape
    C = feature1.shape[1]
    D = qk_w1.shape[0]
    NB = N // BN

    pos1_r = jnp.transpose(pos1, (0, 2, 1))
    pos2_r = jnp.transpose(pos2, (0, 2, 1))
    pos1_raw_r = jnp.transpose(pos1_raw, (0, 2, 1))
    feat1_r = jnp.transpose(feature1, (0, 2, 1))
    feat2_r = jnp.transpose(feature2, (0, 2, 1))

    pe_b1_2 = pe_b1.reshape(1, -1)
    pe_b2_2 = pe_b2.reshape(1, -1)
    qk_b1_2 = qk_b1.reshape(1, -1)
    qk_b2_2 = qk_b2.reshape(1, -1)
    cv_b1_2 = cv_b1.reshape(1, -1)
    cv_b2_2 = cv_b2.reshape(1, -1)

    pos_st = jnp.concatenate([pos1_r, pos2_r], axis=0)
    feat_st = jnp.concatenate([feat1_r, feat2_r], axis=0)
    wspec = lambda shp: pl.BlockSpec(shp, lambda i, j: (0, 0))
    frows = pl.pallas_call(
        _feats_body,
        grid=(2 * B, NB),
        in_specs=[
            pl.BlockSpec((None, BN, 3), lambda i, j: (i, j, 0)),
            pl.BlockSpec((None, BN, C), lambda i, j: (i, j, 0)),
            wspec(pe_w1.shape), wspec(pe_b1_2.shape),
            wspec(pe_w2.shape), wspec(pe_b2_2.shape),
            wspec(qk_w1.shape), wspec(qk_b1_2.shape),
            wspec(qk_w2.shape), wspec(qk_b2_2.shape),
        ],
        out_specs=pl.BlockSpec((None, BN, D), lambda i, j: (i, j, 0)),
        out_shape=jax.ShapeDtypeStruct((2 * B, N, D), jnp.float32),
    )(pos_st, feat_st, pe_w1, pe_b1_2, pe_w2, pe_b2_2, qk_w1, qk_b1_2,
      qk_w2, qk_b2_2)
    f1_rows = frows[:B].reshape(B * N, D)
    f2_rows = frows[B:].reshape(B * N, D)

    idx_loc = pl.pallas_call(
        functools.partial(_knn_body, k=16),
        grid=(B, NB),
        in_specs=[
            pl.BlockSpec((None, BN, 3), lambda b, j: (b, j, 0)),
            pl.BlockSpec((None, 3, N), lambda b, j: (b, 0, 0)),
        ],
        out_specs=pl.BlockSpec((None, BN, 16), lambda b, j: (b, j, 0)),
        out_shape=jax.ShapeDtypeStruct((B, N, 16), jnp.int32),
    )(pos1_r, pos2)

    mult = jnp.asarray(nsample, jnp.int32) // 16
    idx_glob = (jnp.clip(idx_loc * mult, 0, N - 1)
                + (jnp.arange(B, dtype=jnp.int32) * N)[:, None, None])
    idx_flat = idx_glob.reshape(B * N * 16)

    q_st = jnp.concatenate([pos1_raw_r, pos1_r], axis=0)
    r_st = jnp.concatenate([pos1_raw, pos1], axis=0)
    cnet_rows = pl.pallas_call(
        _curv_body,
        grid=(2 * B, N // CBN),
        in_specs=[
            pl.BlockSpec((None, CBN, 3), lambda b, j: (b, j, 0)),
            pl.BlockSpec((None, 3, N), lambda b, j: (b, 0, 0)),
            wspec(cv_w1.shape), wspec(cv_b1_2.shape),
            wspec(cv_w2.shape), wspec(cv_b2_2.shape),
        ],
        out_specs=pl.BlockSpec((None, CBN, 32), lambda b, j: (b, j, 0)),
        out_shape=jax.ShapeDtypeStruct((2 * B, N, 32), jnp.float32),
    )(q_st, r_st, cv_w1, cv_b1_2, cv_w2, cv_b2_2)

    sc_attn = _make_sc_attn(B * N, D, 1.0 / math.sqrt(D))
    cost_rows = sc_attn(f2_rows, f1_rows, idx_flat)

    out = pl.pallas_call(
        _final_body,
        grid=(B, NB),
        in_specs=[
            pl.BlockSpec((None, BN, D), lambda b, j: (b, j, 0)),
            pl.BlockSpec((None, BN, 32), lambda b, j: (b, j, 0)),
            pl.BlockSpec((None, BN, 32), lambda b, j: (b, j, 0)),
            wspec((mlp_w.shape[0], D)),
            wspec((mlp_w.shape[0], 32)),
            wspec((mlp_w.shape[0], 1)),
        ],
        out_specs=pl.BlockSpec((None, mlp_w.shape[0], BN),
                               lambda b, j: (b, 0, j)),
        out_shape=jax.ShapeDtypeStruct((B, mlp_w.shape[0], N), jnp.float32),
    )(cost_rows.reshape(B, N, D), cnet_rows[:B], cnet_rows[B:],
      mlp_w[:, :D], mlp_w[:, D:], mlp_b.reshape(-1, 1))

    return (pos1, out)


# speedup vs baseline: 25.7402x; 1.5222x over previous
import functools
import math

import jax
import jax.numpy as jnp
from jax import lax
from jax.experimental import pallas as pl
from jax.experimental.pallas import tpu as pltpu
from jax.experimental.pallas import tpu_sc as plsc

BN = 512
CBN = 256
RAD2 = 2.5 * 2.5

_DN = (((1,), (1,)), ((), ()))


def _dot(x, w):
    return lax.dot_general(x, w, _DN, preferred_element_type=jnp.float32)


def _feats_body(pos_ref, feat_ref, pw1, pb1, pw2, pb2, qw1, qb1, qw2, qb2,
                out_ref):
    pos = pos_ref[...]
    feat = feat_ref[...]
    h = jnp.maximum(_dot(pos, pw1[...]) + pb1[...], 0.0)
    emb = _dot(h, pw2[...]) + pb2[...]
    f = jnp.concatenate([feat, emb], axis=1)
    h2 = jnp.maximum(_dot(f, qw1[...]) + qb1[...], 0.0)
    out_ref[...] = _dot(h2, qw2[...]) + qb2[...]


def _knn_body(q_ref, rcm_ref, out_ref, *, k):
    q = q_ref[...]
    rcm = rcm_ref[...]
    qq = jnp.sum(q * q, axis=1, keepdims=True)
    rr = jnp.sum(rcm * rcm, axis=0, keepdims=True)
    qr = lax.dot_general(q, rcm, (((1,), (0,)), ((), ())),
                         preferred_element_type=jnp.float32)
    d2 = qq + rr - 2.0 * qr
    iota = lax.broadcasted_iota(jnp.int32, d2.shape, 1)
    big = jnp.int32(d2.shape[1])
    cols = []
    for i in range(k):
        mv = jnp.min(d2, axis=1, keepdims=True)
        hit = d2 == mv
        am = jnp.min(jnp.where(hit, iota, big), axis=1).astype(jnp.int32)
        cols.append(am)
        if i + 1 < k:
            d2 = jnp.where(hit, jnp.inf, d2)
    out_ref[...] = jnp.stack(cols, axis=1)


def _curv_one(q, rcm):
    qq = jnp.sum(q * q, axis=1, keepdims=True)
    rr = jnp.sum(rcm * rcm, axis=0, keepdims=True)
    qr = lax.dot_general(q, rcm, (((1,), (0,)), ((), ())),
                         preferred_element_type=jnp.float32)
    d2 = qq + rr - 2.0 * qr
    d2m = jnp.where(d2 > RAD2, jnp.inf, d2)
    w = d2m
    t = None
    for i in range(10):
        t = jnp.min(w, axis=1, keepdims=True)
        if i + 1 < 10:
            w = jnp.where(w == t, jnp.inf, w)
    sel = ((d2m <= t) & (d2m < jnp.inf)).astype(jnp.float32)
    r4 = jnp.concatenate([rcm, jnp.ones((1, rcm.shape[1]), jnp.float32)], 0)
    acc4 = lax.dot_general(sel, r4, (((1,), (1,)), ((), ())),
                           preferred_element_type=jnp.float32)
    return (acc4[:, :3] - acc4[:, 3:4] * q) / 9.0


def _curv_body(q_ref, rcm_ref, cw1, cb1, cw2, cb2, out_ref):
    def cnet(x):
        h = jnp.maximum(_dot(x, cw1[...]) + cb1[...], 0.0)
        return _dot(h, cw2[...]) + cb2[...]

    out_ref[...] = cnet(_curv_one(q_ref[...], rcm_ref[...]))


def _final_body(cost_ref, cn1_ref, cn2_ref, w1, w2, b, out_ref):
    curv = (cn1_ref[...] - cn2_ref[...]) ** 2
    out_ref[...] = (_dot(w1[...], cost_ref[...])
                    + _dot(w2[...], curv) + b[...])


def _make_sc_attn(total_pts, d, scale):
    NC, NS = 2, 16
    NW = NC * NS
    ppw = total_pts // NW
    CH = 32
    nchunk = ppw // CH
    rows_per_chunk = CH * 16
    ncg = d // 16

    mesh = plsc.VectorSubcoreMesh(core_axis_name="c", subcore_axis_name="s")

    @functools.partial(
        pl.kernel, mesh=mesh,
        out_type=jax.ShapeDtypeStruct((total_pts, d), jnp.float32),
        scratch_types=[
            pltpu.VMEM((rows_per_chunk,), jnp.int32),
            pltpu.VMEM((rows_per_chunk, d), jnp.float32),
            pltpu.VMEM((CH, d), jnp.float32),
            pltpu.VMEM((CH, d), jnp.float32),
            pltpu.SemaphoreType.DMA,
        ],
    )
    def sc_attn(f2_hbm, f1_hbm, idx_hbm, out_hbm, idx_v, rows_v, f1_v, out_v,
                sem):
        wid = lax.axis_index("s") * NC + lax.axis_index("c")

        def chunk_body(ci, carry):
            pbase = wid * ppw + ci * CH
            pltpu.sync_copy(idx_hbm.at[pl.ds(pbase * 16, rows_per_chunk)],
                            idx_v)
            copies = [
                pltpu.async_copy(f2_hbm.at[idx_v.at[pl.ds(j * 128, 128)]],
                                 rows_v.at[pl.ds(j * 128, 128)], sem)
                for j in range(rows_per_chunk // 128)
            ]
            for c in copies:
                c.wait()
            pltpu.sync_copy(f1_hbm.at[pl.ds(pbase, CH)], f1_v)

            def point_body(p, carry2):
                for cg in range(ncg):
                    c0 = cg * 16
                    f1v = f1_v[p, pl.ds(c0, 16)]
                    f1s = f1v * scale
                    f2s = [rows_v[p * 16 + k, pl.ds(c0, 16)]
                           for k in range(16)]
                    a = [f1s * f2 for f2 in f2s]
                    m = a[0]
                    for t in a[1:]:
                        m = jnp.maximum(m, t)
                    e = [jnp.exp(t - m) for t in a]
                    s = e[0]
                    for t in e[1:]:
                        s = s + t
                    num = jnp.zeros((16,), jnp.float32)
                    for k in range(16):
                        dd = f1v - f2s[k]
                        num = num + e[k] * dd * dd
                    out_v[p, pl.ds(c0, 16)] = num / s
                return carry2

            lax.fori_loop(0, CH, point_body, 0)
            pltpu.sync_copy(out_v, out_hbm.at[pl.ds(pbase, CH)])
            return carry

        lax.fori_loop(0, nchunk, chunk_body, 0)

    return sc_attn


def kernel(pos1, pos2, feature1, feature2, nsample, pos1_raw, pe_w1, pe_b1,
           pe_w2, pe_b2, qk_w1, qk_b1, qk_w2, qk_b2, cv_w1, cv_b1, cv_w2,
           cv_b2, mlp_w, mlp_b):
    B, _, N = pos1.shape
    C = feature1.shape[1]
    D = qk_w1.shape[0]
    NB = N // BN

    pos1_r = jnp.transpose(pos1, (0, 2, 1))
    pos2_r = jnp.transpose(pos2, (0, 2, 1))
    pos1_raw_r = jnp.transpose(pos1_raw, (0, 2, 1))
    feat1_r = jnp.transpose(feature1, (0, 2, 1))
    feat2_r = jnp.transpose(feature2, (0, 2, 1))

    pe_b1_2 = pe_b1.reshape(1, -1)
    pe_b2_2 = pe_b2.reshape(1, -1)
    qk_b1_2 = qk_b1.reshape(1, -1)
    qk_b2_2 = qk_b2.reshape(1, -1)
    cv_b1_2 = cv_b1.reshape(1, -1)
    cv_b2_2 = cv_b2.reshape(1, -1)

    pos_st = jnp.concatenate([pos1_r, pos2_r], axis=0)
    feat_st = jnp.concatenate([feat1_r, feat2_r], axis=0)
    wspec = lambda shp: pl.BlockSpec(shp, lambda i, j: (0, 0))
    frows = pl.pallas_call(
        _feats_body,
        grid=(2 * B, NB),
        in_specs=[
            pl.BlockSpec((None, BN, 3), lambda i, j: (i, j, 0)),
            pl.BlockSpec((None, BN, C), lambda i, j: (i, j, 0)),
            wspec(pe_w1.shape), wspec(pe_b1_2.shape),
            wspec(pe_w2.shape), wspec(pe_b2_2.shape),
            wspec(qk_w1.shape), wspec(qk_b1_2.shape),
            wspec(qk_w2.shape), wspec(qk_b2_2.shape),
        ],
        out_specs=pl.BlockSpec((None, BN, D), lambda i, j: (i, j, 0)),
        out_shape=jax.ShapeDtypeStruct((2 * B, N, D), jnp.float32),
    )(pos_st, feat_st, pe_w1, pe_b1_2, pe_w2, pe_b2_2, qk_w1, qk_b1_2,
      qk_w2, qk_b2_2)
    f1_rows = frows[:B].reshape(B * N, D)
    f2_rows = frows[B:].reshape(B * N, D)

    idx_loc = pl.pallas_call(
        functools.partial(_knn_body, k=16),
        grid=(B, NB),
        in_specs=[
            pl.BlockSpec((None, BN, 3), lambda b, j: (b, j, 0)),
            pl.BlockSpec((None, 3, N), lambda b, j: (b, 0, 0)),
        ],
        out_specs=pl.BlockSpec((None, BN, 16), lambda b, j: (b, j, 0)),
        out_shape=jax.ShapeDtypeStruct((B, N, 16), jnp.int32),
    )(pos1_r, pos2)

    mult = jnp.asarray(nsample, jnp.int32) // 16
    idx_glob = (jnp.clip(idx_loc * mult, 0, N - 1)
                + (jnp.arange(B, dtype=jnp.int32) * N)[:, None, None])
    idx_flat = idx_glob.reshape(B * N * 16)

    q_st = jnp.concatenate([pos1_raw_r, pos1_r], axis=0)
    r_st = jnp.concatenate([pos1_raw, pos1], axis=0)
    cnet_rows = pl.pallas_call(
        _curv_body,
        grid=(2 * B, N // CBN),
        in_specs=[
            pl.BlockSpec((None, CBN, 3), lambda b, j: (b, j, 0)),
            pl.BlockSpec((None, 3, N), lambda b, j: (b, 0, 0)),
            wspec(cv_w1.shape), wspec(cv_b1_2.shape),
            wspec(cv_w2.shape), wspec(cv_b2_2.shape),
        ],
        out_specs=pl.BlockSpec((None, CBN, 32), lambda b, j: (b, j, 0)),
        out_shape=jax.ShapeDtypeStruct((2 * B, N, 32), jnp.float32),
    )(q_st, r_st, cv_w1, cv_b1_2, cv_w2, cv_b2_2)

    sc_attn = _make_sc_attn(B * N, D, 1.0 / math.sqrt(D))
    cost_rows = sc_attn(f2_rows, f1_rows, idx_flat)

    out = pl.pallas_call(
        _final_body,
        grid=(B, NB),
        in_specs=[
            pl.BlockSpec((None, BN, D), lambda b, j: (b, j, 0)),
            pl.BlockSpec((None, BN, 32), lambda b, j: (b, j, 0)),
            pl.BlockSpec((None, BN, 32), lambda b, j: (b, j, 0)),
            wspec((mlp_w.shape[0], D)),
            wspec((mlp_w.shape[0], 32)),
            wspec((mlp_w.shape[0], 1)),
        ],
        out_specs=pl.BlockSpec((None, mlp_w.shape[0], BN),
                               lambda b, j: (b, 0, j)),
        out_shape=jax.ShapeDtypeStruct((B, mlp_w.shape[0], N), jnp.float32),
    )(cost_rows.reshape(B, N, D), cnet_rows[:B], cnet_rows[B:],
      mlp_w[:, :D], mlp_w[:, D:], mlp_b.reshape(-1, 1))

    return (pos1, out)


# speedup vs baseline: 26.4139x; 1.0262x over previous
import functools
import math

import jax
import jax.numpy as jnp
from jax import lax
from jax.experimental import pallas as pl
from jax.experimental.pallas import tpu as pltpu
from jax.experimental.pallas import tpu_sc as plsc

BN = 512
CBN = 512
RAD2 = 2.5 * 2.5

_DN = (((1,), (1,)), ((), ()))


def _dot(x, w):
    return lax.dot_general(x, w, _DN, preferred_element_type=jnp.float32)


def _feats_body(pos_ref, feat_ref, pw1, pb1, pw2, pb2, qw1, qb1, qw2, qb2,
                out_ref):
    pos = pos_ref[...]
    feat = feat_ref[...]
    h = jnp.maximum(_dot(pos, pw1[...]) + pb1[...], 0.0)
    emb = _dot(h, pw2[...]) + pb2[...]
    f = jnp.concatenate([feat, emb], axis=1)
    h2 = jnp.maximum(_dot(f, qw1[...]) + qb1[...], 0.0)
    out_ref[...] = _dot(h2, qw2[...]) + qb2[...]


def _knn_body(q_ref, rcm_ref, out_ref, *, k):
    q = q_ref[...]
    rcm = rcm_ref[...]
    qq = jnp.sum(q * q, axis=1, keepdims=True)
    rr = jnp.sum(rcm * rcm, axis=0, keepdims=True)
    qr = lax.dot_general(q, rcm, (((1,), (0,)), ((), ())),
                         preferred_element_type=jnp.float32)
    d2 = qq + rr - 2.0 * qr
    iota = lax.broadcasted_iota(jnp.int32, d2.shape, 1)
    big = jnp.int32(d2.shape[1])
    cols = []
    for i in range(k):
        mv = jnp.min(d2, axis=1, keepdims=True)
        hit = d2 == mv
        am = jnp.min(jnp.where(hit, iota, big), axis=1).astype(jnp.int32)
        cols.append(am)
        if i + 1 < k:
            d2 = jnp.where(hit, jnp.inf, d2)
    out_ref[...] = jnp.stack(cols, axis=1)


def _curv_one(q, rcm):
    qq = jnp.sum(q * q, axis=1, keepdims=True)
    rr = jnp.sum(rcm * rcm, axis=0, keepdims=True)
    qr = lax.dot_general(q, rcm, (((1,), (0,)), ((), ())),
                         preferred_element_type=jnp.float32)
    d2 = qq + rr - 2.0 * qr
    d2m = jnp.where(d2 > RAD2, jnp.inf, d2)
    w = d2m
    t = None
    for i in range(10):
        t = jnp.min(w, axis=1, keepdims=True)
        if i + 1 < 10:
            w = jnp.where(w == t, jnp.inf, w)
    sel = ((d2m <= t) & (d2m < jnp.inf)).astype(jnp.float32)
    r4 = jnp.concatenate([rcm, jnp.ones((1, rcm.shape[1]), jnp.float32)], 0)
    acc4 = lax.dot_general(sel, r4, (((1,), (1,)), ((), ())),
                           preferred_element_type=jnp.float32)
    return (acc4[:, :3] - acc4[:, 3:4] * q) / 9.0


def _curv_body(q_ref, rcm_ref, cw1, cb1, cw2, cb2, out_ref):
    def cnet(x):
        h = jnp.maximum(_dot(x, cw1[...]) + cb1[...], 0.0)
        return _dot(h, cw2[...]) + cb2[...]

    out_ref[...] = cnet(_curv_one(q_ref[...], rcm_ref[...]))


def _final_body(cost_ref, cn1_ref, cn2_ref, w1, w2, b, out_ref):
    curv = (cn1_ref[...] - cn2_ref[...]) ** 2
    out_ref[...] = (_dot(w1[...], cost_ref[...])
                    + _dot(w2[...], curv) + b[...])


def _make_sc_attn(total_pts, d, scale):
    NC, NS = 2, 16
    NW = NC * NS
    ppw = total_pts // NW
    CH = 32
    nchunk = ppw // CH
    rows_per_chunk = CH * 16
    ncg = d // 16

    mesh = plsc.VectorSubcoreMesh(core_axis_name="c", subcore_axis_name="s")

    @functools.partial(
        pl.kernel, mesh=mesh,
        out_type=jax.ShapeDtypeStruct((total_pts, d), jnp.float32),
        scratch_types=[
            pltpu.VMEM((rows_per_chunk,), jnp.int32),
            pltpu.VMEM((rows_per_chunk, d), jnp.float32),
            pltpu.VMEM((CH, d), jnp.float32),
            pltpu.VMEM((CH, d), jnp.float32),
            pltpu.SemaphoreType.DMA,
        ],
    )
    def sc_attn(f2_hbm, f1_hbm, idx_hbm, out_hbm, idx_v, rows_v, f1_v, out_v,
                sem):
        wid = lax.axis_index("s") * NC + lax.axis_index("c")

        def chunk_body(ci, carry):
            pbase = wid * ppw + ci * CH
            pltpu.sync_copy(idx_hbm.at[pl.ds(pbase * 16, rows_per_chunk)],
                            idx_v)
            copies = [
                pltpu.async_copy(f2_hbm.at[idx_v.at[pl.ds(j * 128, 128)]],
                                 rows_v.at[pl.ds(j * 128, 128)], sem)
                for j in range(rows_per_chunk // 128)
            ]
            for c in copies:
                c.wait()
            pltpu.sync_copy(f1_hbm.at[pl.ds(pbase, CH)], f1_v)

            def point_body(p, carry2):
                for cg in range(ncg):
                    c0 = cg * 16
                    f1v = f1_v[p, pl.ds(c0, 16)]
                    f1s = f1v * scale
                    f2s = [rows_v[p * 16 + k, pl.ds(c0, 16)]
                           for k in range(16)]
                    a = [f1s * f2 for f2 in f2s]
                    m = a[0]
                    for t in a[1:]:
                        m = jnp.maximum(m, t)
                    e = [jnp.exp(t - m) for t in a]
                    s = e[0]
                    for t in e[1:]:
                        s = s + t
                    num = jnp.zeros((16,), jnp.float32)
                    for k in range(16):
                        dd = f1v - f2s[k]
                        num = num + e[k] * dd * dd
                    out_v[p, pl.ds(c0, 16)] = num / s
                return carry2

            lax.fori_loop(0, CH, point_body, 0)
            pltpu.sync_copy(out_v, out_hbm.at[pl.ds(pbase, CH)])
            return carry

        lax.fori_loop(0, nchunk, chunk_body, 0)

    return sc_attn


def kernel(pos1, pos2, feature1, feature2, nsample, pos1_raw, pe_w1, pe_b1,
           pe_w2, pe_b2, qk_w1, qk_b1, qk_w2, qk_b2, cv_w1, cv_b1, cv_w2,
           cv_b2, mlp_w, mlp_b):
    B, _, N = pos1.shape
    C = feature1.shape[1]
    D = qk_w1.shape[0]
    NB = N // BN

    pos1_r = jnp.transpose(pos1, (0, 2, 1))
    pos2_r = jnp.transpose(pos2, (0, 2, 1))
    pos1_raw_r = jnp.transpose(pos1_raw, (0, 2, 1))
    feat1_r = jnp.transpose(feature1, (0, 2, 1))
    feat2_r = jnp.transpose(feature2, (0, 2, 1))

    pe_b1_2 = pe_b1.reshape(1, -1)
    pe_b2_2 = pe_b2.reshape(1, -1)
    qk_b1_2 = qk_b1.reshape(1, -1)
    qk_b2_2 = qk_b2.reshape(1, -1)
    cv_b1_2 = cv_b1.reshape(1, -1)
    cv_b2_2 = cv_b2.reshape(1, -1)

    pos_st = jnp.concatenate([pos1_r, pos2_r], axis=0)
    feat_st = jnp.concatenate([feat1_r, feat2_r], axis=0)
    wspec = lambda shp: pl.BlockSpec(shp, lambda i, j: (0, 0))
    frows = pl.pallas_call(
        _feats_body,
        grid=(2 * B, NB),
        in_specs=[
            pl.BlockSpec((None, BN, 3), lambda i, j: (i, j, 0)),
            pl.BlockSpec((None, BN, C), lambda i, j: (i, j, 0)),
            wspec(pe_w1.shape), wspec(pe_b1_2.shape),
            wspec(pe_w2.shape), wspec(pe_b2_2.shape),
            wspec(qk_w1.shape), wspec(qk_b1_2.shape),
            wspec(qk_w2.shape), wspec(qk_b2_2.shape),
        ],
        out_specs=pl.BlockSpec((None, BN, D), lambda i, j: (i, j, 0)),
        out_shape=jax.ShapeDtypeStruct((2 * B, N, D), jnp.float32),
    )(pos_st, feat_st, pe_w1, pe_b1_2, pe_w2, pe_b2_2, qk_w1, qk_b1_2,
      qk_w2, qk_b2_2)
    f1_rows = frows[:B].reshape(B * N, D)
    f2_rows = frows[B:].reshape(B * N, D)

    idx_loc = pl.pallas_call(
        functools.partial(_knn_body, k=16),
        grid=(B, NB),
        in_specs=[
            pl.BlockSpec((None, BN, 3), lambda b, j: (b, j, 0)),
            pl.BlockSpec((None, 3, N), lambda b, j: (b, 0, 0)),
        ],
        out_specs=pl.BlockSpec((None, BN, 16), lambda b, j: (b, j, 0)),
        out_shape=jax.ShapeDtypeStruct((B, N, 16), jnp.int32),
    )(pos1_r, pos2)

    mult = jnp.asarray(nsample, jnp.int32) // 16
    idx_glob = (jnp.clip(idx_loc * mult, 0, N - 1)
                + (jnp.arange(B, dtype=jnp.int32) * N)[:, None, None])
    idx_flat = idx_glob.reshape(B * N * 16)

    q_st = jnp.concatenate([pos1_raw_r, pos1_r], axis=0)
    r_st = jnp.concatenate([pos1_raw, pos1], axis=0)
    cnet_rows = pl.pallas_call(
        _curv_body,
        grid=(2 * B, N // CBN),
        in_specs=[
            pl.BlockSpec((None, CBN, 3), lambda b, j: (b, j, 0)),
            pl.BlockSpec((None, 3, N), lambda b, j: (b, 0, 0)),
            wspec(cv_w1.shape), wspec(cv_b1_2.shape),
            wspec(cv_w2.shape), wspec(cv_b2_2.shape),
        ],
        out_specs=pl.BlockSpec((None, CBN, 32), lambda b, j: (b, j, 0)),
        out_shape=jax.ShapeDtypeStruct((2 * B, N, 32), jnp.float32),
    )(q_st, r_st, cv_w1, cv_b1_2, cv_w2, cv_b2_2)

    sc_attn = _make_sc_attn(B * N, D, 1.0 / math.sqrt(D))
    cost_rows = sc_attn(f2_rows, f1_rows, idx_flat)

    out = pl.pallas_call(
        _final_body,
        grid=(B, NB),
        in_specs=[
            pl.BlockSpec((None, BN, D), lambda b, j: (b, j, 0)),
            pl.BlockSpec((None, BN, 32), lambda b, j: (b, j, 0)),
            pl.BlockSpec((None, BN, 32), lambda b, j: (b, j, 0)),
            wspec((mlp_w.shape[0], D)),
            wspec((mlp_w.shape[0], 32)),
            wspec((mlp_w.shape[0], 1)),
        ],
        out_specs=pl.BlockSpec((None, mlp_w.shape[0], BN),
                               lambda b, j: (b, 0, j)),
        out_shape=jax.ShapeDtypeStruct((B, mlp_w.shape[0], N), jnp.float32),
    )(cost_rows.reshape(B, N, D), cnet_rows[:B], cnet_rows[B:],
      mlp_w[:, :D], mlp_w[:, D:], mlp_b.reshape(-1, 1))

    return (pos1, out)


# speedup vs baseline: 27.3138x; 1.0341x over previous
import functools
import math

import jax
import jax.numpy as jnp
from jax import lax
from jax.experimental import pallas as pl
from jax.experimental.pallas import tpu as pltpu
from jax.experimental.pallas import tpu_sc as plsc

BN = 512
CBN = 512
RAD2 = 2.5 * 2.5

_DN = (((1,), (1,)), ((), ()))


def _dot(x, w):
    return lax.dot_general(x, w, _DN, preferred_element_type=jnp.float32)


def _feats_body(pos_ref, feat_ref, pw1, pb1, pw2, pb2, qw1, qb1, qw2, qb2,
                out_ref):
    pos = pos_ref[...]
    feat = feat_ref[...]
    h = jnp.maximum(_dot(pos, pw1[...]) + pb1[...], 0.0)
    emb = _dot(h, pw2[...]) + pb2[...]
    f = jnp.concatenate([feat, emb], axis=1)
    h2 = jnp.maximum(_dot(f, qw1[...]) + qb1[...], 0.0)
    out_ref[...] = _dot(h2, qw2[...]) + qb2[...]


def _knn_body(q_ref, rcm_ref, out_ref, *, k):
    q = q_ref[...]
    rcm = rcm_ref[...]
    qq = jnp.sum(q * q, axis=1, keepdims=True)
    rr = jnp.sum(rcm * rcm, axis=0, keepdims=True)
    qr = lax.dot_general(q, rcm, (((1,), (0,)), ((), ())),
                         preferred_element_type=jnp.float32)
    d2 = qq + rr - 2.0 * qr
    iota = lax.broadcasted_iota(jnp.int32, d2.shape, 1)
    big = jnp.int32(d2.shape[1])
    cols = []
    for i in range(k):
        am = jnp.argmin(d2, axis=1).astype(jnp.int32)
        cols.append(am)
        if i + 1 < k:
            d2 = jnp.where(iota == am[:, None], jnp.inf, d2)
    out_ref[...] = jnp.stack(cols, axis=1)


def _curv_one(q, rcm):
    qq = jnp.sum(q * q, axis=1, keepdims=True)
    rr = jnp.sum(rcm * rcm, axis=0, keepdims=True)
    qr = lax.dot_general(q, rcm, (((1,), (0,)), ((), ())),
                         preferred_element_type=jnp.float32)
    d2 = qq + rr - 2.0 * qr
    d2m = jnp.where(d2 > RAD2, jnp.inf, d2)
    w = d2m
    t = None
    for i in range(10):
        t = jnp.min(w, axis=1, keepdims=True)
        if i + 1 < 10:
            w = jnp.where(w == t, jnp.inf, w)
    sel = ((d2m <= t) & (d2m < jnp.inf)).astype(jnp.float32)
    r4 = jnp.concatenate([rcm, jnp.ones((1, rcm.shape[1]), jnp.float32)], 0)
    acc4 = lax.dot_general(sel, r4, (((1,), (1,)), ((), ())),
                           preferred_element_type=jnp.float32)
    return (acc4[:, :3] - acc4[:, 3:4] * q) / 9.0


def _curv_body(q_ref, rcm_ref, cw1, cb1, cw2, cb2, out_ref):
    def cnet(x):
        h = jnp.maximum(_dot(x, cw1[...]) + cb1[...], 0.0)
        return _dot(h, cw2[...]) + cb2[...]

    out_ref[...] = cnet(_curv_one(q_ref[...], rcm_ref[...]))


def _final_body(cost_ref, cn1_ref, cn2_ref, w1, w2, b, out_ref):
    curv = (cn1_ref[...] - cn2_ref[...]) ** 2
    out_ref[...] = (_dot(w1[...], cost_ref[...])
                    + _dot(w2[...], curv) + b[...])


def _make_sc_attn(total_pts, d, scale):
    NC, NS = 2, 16
    NW = NC * NS
    ppw = total_pts // NW
    CH = 32
    nchunk = ppw // CH
    rows_per_chunk = CH * 16
    ncg = d // 16

    mesh = plsc.VectorSubcoreMesh(core_axis_name="c", subcore_axis_name="s")

    @functools.partial(
        pl.kernel, mesh=mesh,
        out_type=jax.ShapeDtypeStruct((total_pts, d), jnp.float32),
        scratch_types=[
            pltpu.VMEM((rows_per_chunk,), jnp.int32),
            pltpu.VMEM((rows_per_chunk, d), jnp.float32),
            pltpu.VMEM((CH, d), jnp.float32),
            pltpu.VMEM((CH, d), jnp.float32),
            pltpu.SemaphoreType.DMA,
        ],
    )
    def sc_attn(f2_hbm, f1_hbm, idx_hbm, out_hbm, idx_v, rows_v, f1_v, out_v,
                sem):
        wid = lax.axis_index("s") * NC + lax.axis_index("c")

        def chunk_body(ci, carry):
            pbase = wid * ppw + ci * CH
            pltpu.sync_copy(idx_hbm.at[pl.ds(pbase * 16, rows_per_chunk)],
                            idx_v)
            copies = [
                pltpu.async_copy(f2_hbm.at[idx_v.at[pl.ds(j * 128, 128)]],
                                 rows_v.at[pl.ds(j * 128, 128)], sem)
                for j in range(rows_per_chunk // 128)
            ]
            for c in copies:
                c.wait()
            pltpu.sync_copy(f1_hbm.at[pl.ds(pbase, CH)], f1_v)

            def point_body(p, carry2):
                for cg in range(ncg):
                    c0 = cg * 16
                    f1v = f1_v[p, pl.ds(c0, 16)]
                    f1s = f1v * scale
                    f2s = [rows_v[p * 16 + k, pl.ds(c0, 16)]
                           for k in range(16)]
                    a = [f1s * f2 for f2 in f2s]
                    m = a[0]
                    for t in a[1:]:
                        m = jnp.maximum(m, t)
                    e = [jnp.exp(t - m) for t in a]
                    s = e[0]
                    for t in e[1:]:
                        s = s + t
                    num = jnp.zeros((16,), jnp.float32)
                    for k in range(16):
                        dd = f1v - f2s[k]
                        num = num + e[k] * dd * dd
                    out_v[p, pl.ds(c0, 16)] = num / s
                return carry2

            lax.fori_loop(0, CH, point_body, 0)
            pltpu.sync_copy(out_v, out_hbm.at[pl.ds(pbase, CH)])
            return carry

        lax.fori_loop(0, nchunk, chunk_body, 0)

    return sc_attn


def kernel(pos1, pos2, feature1, feature2, nsample, pos1_raw, pe_w1, pe_b1,
           pe_w2, pe_b2, qk_w1, qk_b1, qk_w2, qk_b2, cv_w1, cv_b1, cv_w2,
           cv_b2, mlp_w, mlp_b):
    B, _, N = pos1.shape
    C = feature1.shape[1]
    D = qk_w1.shape[0]
    NB = N // BN

    pos1_r = jnp.transpose(pos1, (0, 2, 1))
    pos2_r = jnp.transpose(pos2, (0, 2, 1))
    pos1_raw_r = jnp.transpose(pos1_raw, (0, 2, 1))
    feat1_r = jnp.transpose(feature1, (0, 2, 1))
    feat2_r = jnp.transpose(feature2, (0, 2, 1))

    pe_b1_2 = pe_b1.reshape(1, -1)
    pe_b2_2 = pe_b2.reshape(1, -1)
    qk_b1_2 = qk_b1.reshape(1, -1)
    qk_b2_2 = qk_b2.reshape(1, -1)
    cv_b1_2 = cv_b1.reshape(1, -1)
    cv_b2_2 = cv_b2.reshape(1, -1)

    pos_st = jnp.concatenate([pos1_r, pos2_r], axis=0)
    feat_st = jnp.concatenate([feat1_r, feat2_r], axis=0)
    wspec = lambda shp: pl.BlockSpec(shp, lambda i, j: (0, 0))
    frows = pl.pallas_call(
        _feats_body,
        grid=(2 * B, NB),
        in_specs=[
            pl.BlockSpec((None, BN, 3), lambda i, j: (i, j, 0)),
            pl.BlockSpec((None, BN, C), lambda i, j: (i, j, 0)),
            wspec(pe_w1.shape), wspec(pe_b1_2.shape),
            wspec(pe_w2.shape), wspec(pe_b2_2.shape),
            wspec(qk_w1.shape), wspec(qk_b1_2.shape),
            wspec(qk_w2.shape), wspec(qk_b2_2.shape),
        ],
        out_specs=pl.BlockSpec((None, BN, D), lambda i, j: (i, j, 0)),
        out_shape=jax.ShapeDtypeStruct((2 * B, N, D), jnp.float32),
    )(pos_st, feat_st, pe_w1, pe_b1_2, pe_w2, pe_b2_2, qk_w1, qk_b1_2,
      qk_w2, qk_b2_2)
    f1_rows = frows[:B].reshape(B * N, D)
    f2_rows = frows[B:].reshape(B * N, D)

    idx_loc = pl.pallas_call(
        functools.partial(_knn_body, k=16),
        grid=(B, NB),
        in_specs=[
            pl.BlockSpec((None, BN, 3), lambda b, j: (b, j, 0)),
            pl.BlockSpec((None, 3, N), lambda b, j: (b, 0, 0)),
        ],
        out_specs=pl.BlockSpec((None, BN, 16), lambda b, j: (b, j, 0)),
        out_shape=jax.ShapeDtypeStruct((B, N, 16), jnp.int32),
    )(pos1_r, pos2)

    mult = jnp.asarray(nsample, jnp.int32) // 16
    idx_glob = (jnp.clip(idx_loc * mult, 0, N - 1)
                + (jnp.arange(B, dtype=jnp.int32) * N)[:, None, None])
    idx_flat = idx_glob.reshape(B * N * 16)

    q_st = jnp.concatenate([pos1_raw_r, pos1_r], axis=0)
    r_st = jnp.concatenate([pos1_raw, pos1], axis=0)
    cnet_rows = pl.pallas_call(
        _curv_body,
        grid=(2 * B, N // CBN),
        in_specs=[
            pl.BlockSpec((None, CBN, 3), lambda b, j: (b, j, 0)),
            pl.BlockSpec((None, 3, N), lambda b, j: (b, 0, 0)),
            wspec(cv_w1.shape), wspec(cv_b1_2.shape),
            wspec(cv_w2.shape), wspec(cv_b2_2.shape),
        ],
        out_specs=pl.BlockSpec((None, CBN, 32), lambda b, j: (b, j, 0)),
        out_shape=jax.ShapeDtypeStruct((2 * B, N, 32), jnp.float32),
    )(q_st, r_st, cv_w1, cv_b1_2, cv_w2, cv_b2_2)

    sc_attn = _make_sc_attn(B * N, D, 1.0 / math.sqrt(D))
    cost_rows = sc_attn(f2_rows, f1_rows, idx_flat)

    out = pl.pallas_call(
        _final_body,
        grid=(B, NB),
        in_specs=[
            pl.BlockSpec((None, BN, D), lambda b, j: (b, j, 0)),
            pl.BlockSpec((None, BN, 32), lambda b, j: (b, j, 0)),
            pl.BlockSpec((None, BN, 32), lambda b, j: (b, j, 0)),
            wspec((mlp_w.shape[0], D)),
            wspec((mlp_w.shape[0], 32)),
            wspec((mlp_w.shape[0], 1)),
        ],
        out_specs=pl.BlockSpec((None, mlp_w.shape[0], BN),
                               lambda b, j: (b, 0, j)),
        out_shape=jax.ShapeDtypeStruct((B, mlp_w.shape[0], N), jnp.float32),
    )(cost_rows.reshape(B, N, D), cnet_rows[:B], cnet_rows[B:],
      mlp_w[:, :D], mlp_w[:, D:], mlp_b.reshape(-1, 1))

    return (pos1, out)
